# baseline (speedup 1.0000x reference)
import functools

import jax
import jax.numpy as jnp
from jax import lax
from jax.experimental import pallas as pl
from jax.experimental.pallas import tpu as pltpu
from jax.experimental.pallas import tpu_sc as plsc

VOCAB = 100000
HIDDEN = 128
BATCH = 1024
SEQ = 200

NC = 2
NS = 16
NW = NC * NS

CHUNK = 100
B_PER_W = BATCH // NW
N_CHUNKS = B_PER_W * SEQ // CHUNK
HALVES = SEQ // CHUNK
LANES = 16
NGRP = HIDDEN // LANES


def _sc_gather_pool(idx3, table, w_t, bias):
  mesh = plsc.VectorSubcoreMesh(
      core_axis_name="c", subcore_axis_name="s", num_cores=NC, num_subcores=NS
  )
  nbuf = 4
  assert B_PER_W % (2 * nbuf) == 0

  @functools.partial(
      pl.kernel,
      out_type=(
          jax.ShapeDtypeStruct((BATCH, SEQ, HIDDEN), jnp.float32),
          jax.ShapeDtypeStruct((BATCH, HIDDEN), jnp.float32),
      ),
      mesh=mesh,
      scratch_types=[
          pltpu.VMEM((N_CHUNKS, CHUNK), jnp.int32),
          pltpu.VMEM((nbuf, SEQ, HIDDEN), jnp.float32),
          pltpu.VMEM((HIDDEN, HIDDEN), jnp.float32),
          pltpu.VMEM((nbuf, HIDDEN), jnp.float32),
          pltpu.VMEM((2 * nbuf, HIDDEN), jnp.float32),
          [pltpu.SemaphoreType.DMA] * nbuf,
          [pltpu.SemaphoreType.DMA] * nbuf,
      ],
  )
  def gather_kernel(idx_hbm, table_hbm, wt_hbm, bias_hbm, out_hbm, pooled_hbm,
                    idx_v, rows_v, wt_v, x_v, pooled_v, gsems, ssems):
    wid = lax.axis_index("s") * NC + lax.axis_index("c")
    pltpu.sync_copy(idx_hbm.at[wid], idx_v)
    pltpu.sync_copy(wt_hbm, wt_v)
    pltpu.sync_copy(bias_hbm, x_v.at[0])
    bias_vecs = [x_v[0, pl.ds(g * LANES, LANES)] for g in range(NGRP)]

    def start_gathers(i, b):
      for h in range(HALVES):
        pltpu.async_copy(table_hbm.at[idx_v.at[HALVES * i + h]],
                         rows_v.at[b, pl.ds(h * CHUNK, CHUNK)], gsems[b])

    def wait_gathers(i, b):
      for h in range(HALVES):
        pltpu.make_async_copy(table_hbm.at[idx_v.at[HALVES * i + h]],
                              rows_v.at[b, pl.ds(h * CHUNK, CHUNK)],
                              gsems[b]).wait()

    def store_sem_op(i, b):
      row = wid * B_PER_W + i
      return (
          pltpu.make_async_copy(rows_v.at[b, pl.ds(0, 104)],
                                out_hbm.at[row, pl.ds(0, 104)], ssems[b]),
          pltpu.make_async_copy(rows_v.at[b, pl.ds(104, 96)],
                                out_hbm.at[row, pl.ds(104, 96)], ssems[b]),
      )

    for b in range(nbuf):
      start_gathers(b, b)

    def group(t, _):
      for b in range(nbuf):
        i = nbuf * t + b
        wait_gathers(i, b)
        for g in range(NGRP):
          x_v[b, pl.ds(g * LANES, LANES)] = rows_v[b, 0,
                                                   pl.ds(g * LANES, LANES)]
        for op in store_sem_op(i, b):
          op.start()

      for b in range(nbuf):
        i = nbuf * t + b + nbuf

        @pl.when(i < B_PER_W)
        def _():
          for op in store_sem_op(i - nbuf, b):
            op.wait()
          start_gathers(i, b)

      for b in range(nbuf):
        prow = (t % 2) * nbuf + b

        def kstep(k, acc):
          xv = x_v[b, pl.ds(k * LANES, LANES)]
          for l in range(LANES):
            xh = xv[l]
            acc = tuple(
                acc[g] + xh * wt_v[k * LANES + l, pl.ds(g * LANES, LANES)]
                for g in range(NGRP)
            )
          return acc

        acc = lax.fori_loop(0, NGRP, kstep, tuple(bias_vecs))
        for g in range(NGRP):
          pooled_v[prow, pl.ds(g * LANES, LANES)] = acc[g]

      @pl.when(t % 2 == 1)
      def _():
        off = pl.multiple_of(
            wid * B_PER_W + lax.div(t - 1, 2) * (2 * nbuf), 2 * nbuf)
        pltpu.sync_copy(pooled_v, pooled_hbm.at[pl.ds(off, 2 * nbuf)])

      return 0

    lax.fori_loop(0, B_PER_W // nbuf, group, 0)

    for b in range(nbuf):
      for op in store_sem_op(B_PER_W - nbuf + b, b):
        op.wait()

  return gather_kernel(idx3, table, w_t, bias)


def kernel(input_ids, emb_table, pooler_w, pooler_b):
  idx3 = input_ids.reshape(NW, N_CHUNKS, CHUNK).astype(jnp.int32)
  seq3, pooled = _sc_gather_pool(idx3, emb_table, pooler_w.T, pooler_b)
  return (seq3, pooled)
